# baseline (speedup 1.0000x reference)
import functools

import jax
import jax.numpy as jnp
from jax import lax
from jax.experimental import pallas as pl
from jax.experimental.pallas import tpu as pltpu
from jax.experimental.pallas import tpu_sc as plsc

N = 100000
EMB = 128
BATCH = 1024
OUT = 1

TCHUNK = 800
G_TC = 25
N_SC = N - G_TC * TCHUNK
TC_OFF = N_SC // TCHUNK

NC = 2
NS = 16
NW = NC * NS
CHUNK = 128
NBUF = 6
ROUNDS = 20
ACC_ROWS = 1152
ZROWS = ACC_ROWS // NS

_BN_INV = 0.9999950000374997


@functools.cache
def _make_sc_segment_sum():
    mesh = plsc.VectorSubcoreMesh(
        core_axis_name="c", subcore_axis_name="s",
        num_cores=NC, num_subcores=NS)
    return pl.kernel(
        _sc_segment_sum_body,
        out_type=jax.ShapeDtypeStruct((NC, BATCH, EMB), jnp.float32),
        mesh=mesh,
        scratch_types=[
            pltpu.VMEM((ROUNDS, CHUNK), jnp.int32),
            [pltpu.VMEM((CHUNK, EMB), jnp.float32) for _ in range(NBUF)],
            pltpu.VMEM((ZROWS, EMB), jnp.float32),
            pltpu.VMEM_SHARED((ACC_ROWS, EMB), jnp.float32),
            [pltpu.SemaphoreType.DMA for _ in range(NBUF)],
            [pltpu.SemaphoreType.DMA for _ in range(NBUF)],
            pltpu.SemaphoreType.DMA,
        ],
    )


def _sc_segment_sum_body(x_hbm, idx_hbm, out_hbm,
                         idx_v, bufs, bounce_v, acc_sh, lsems, ssems, isem):
    c = lax.axis_index("c")
    s = lax.axis_index("s")
    w = c * NS + s

    def row_start(r):
        return (w * ROUNDS + r) * CHUNK

    def cond_full(r):
        return row_start(r) + CHUNK <= N_SC

    def load_full(r):
        return pltpu.make_async_copy(
            x_hbm.at[pl.ds(row_start(r), CHUNK)], bufs[r % NBUF],
            lsems[r % NBUF])

    def scat(r):
        return pltpu.make_async_copy(
            bufs[r % NBUF], acc_sh.at[idx_v.at[r]], ssems[r % NBUF])

    def issue_load(r):
        @pl.when(cond_full(r))
        def _():
            load_full(r).start()

    def idx_full(r):
        return pltpu.make_async_copy(
            idx_hbm.at[pl.ds(row_start(r), CHUNK)], idx_v.at[r], isem)

    for k in range(NBUF - 1):
        issue_load(k)

    for r in range(ROUNDS):
        @pl.when(cond_full(r))
        def _():
            idx_full(r).start()

    zvec = jnp.zeros((16,), jnp.float32)

    def _zero_row(i, _):
        for j in range(EMB // 16):
            bounce_v[i, pl.ds(j * 16, 16)] = zvec
        return 0

    lax.fori_loop(0, ZROWS, _zero_row, 0)
    pltpu.sync_copy(bounce_v, acc_sh.at[pl.ds(s * ZROWS, ZROWS)])

    for r in range(ROUNDS):
        @pl.when(cond_full(r))
        def _():
            idx_full(r).wait()

    plsc.subcore_barrier()

    for r in range(ROUNDS):
        @pl.when(cond_full(r))
        def _full():
            load_full(r).wait()
            scat(r).start(add=True)

        nxt = r + NBUF - 1
        if nxt < ROUNDS:
            prev = nxt - NBUF
            if prev >= 0:
                @pl.when(cond_full(prev))
                def _drain():
                    scat(prev).wait()
            issue_load(nxt)

    for r in range(max(0, ROUNDS - NBUF), ROUNDS):
        @pl.when(cond_full(r))
        def _drain_tail():
            scat(r).wait()

    plsc.subcore_barrier()

    wrows = BATCH // NS
    pltpu.sync_copy(acc_sh.at[pl.ds(s * wrows, wrows)],
                    bounce_v.at[pl.ds(0, wrows)])
    pltpu.sync_copy(bounce_v.at[pl.ds(0, wrows)],
                    out_hbm.at[c].at[pl.ds(s * wrows, wrows)])


def _tc_tail_body(x_ref, idx_ref, o_ref):
    g = pl.program_id(0)

    @pl.when(g == 0)
    def _():
        o_ref[...] = jnp.zeros_like(o_ref)

    ids = idx_ref[0, 0, :]
    oht = (lax.broadcasted_iota(jnp.int32, (BATCH, TCHUNK), 0)
           == ids[None, :]).astype(jnp.bfloat16)
    xs = x_ref[...]
    hi = xs.astype(jnp.bfloat16)
    lo = (xs - hi.astype(jnp.float32)).astype(jnp.bfloat16)
    o_ref[...] += (jnp.dot(oht, hi, preferred_element_type=jnp.float32)
                   + jnp.dot(oht, lo, preferred_element_type=jnp.float32))


def _tc_tail_partial(x_0, idx3):
    return pl.pallas_call(
        _tc_tail_body,
        grid=(G_TC,),
        in_specs=[
            pl.BlockSpec((TCHUNK, EMB), lambda g: (TC_OFF + g, 0)),
            pl.BlockSpec((1, 1, TCHUNK), lambda g: (TC_OFF + g, 0, 0)),
        ],
        out_specs=pl.BlockSpec((BATCH, EMB), lambda g: (0, 0)),
        out_shape=jax.ShapeDtypeStruct((BATCH, EMB), jnp.float32),
    )(x_0, idx3)


def _head_body(p_ref, ptc_ref, w1, b1, g1, bt1, w2, b2, g2, bt2, w3t, b3, o_ref):
    pooled = p_ref[0] + p_ref[1] + ptc_ref[...]
    h = jnp.dot(pooled, w1[...], preferred_element_type=jnp.float32) + b1[...]
    h = jnp.maximum(h * (g1[...] * _BN_INV) + bt1[...], 0.0)
    h = jnp.dot(h, w2[...], preferred_element_type=jnp.float32) + b2[...]
    h = jnp.maximum(h * (g2[...] * _BN_INV) + bt2[...], 0.0)
    o_ref[...] = jnp.sum(h * w3t[...], axis=1, keepdims=True) + b3[...]


def _head(partials, ptc, W1, b1, g1, bt1, W2, b2, g2, bt2, W3, b3):
    row = lambda v: v.reshape(1, -1)
    return pl.pallas_call(
        _head_body,
        out_shape=jax.ShapeDtypeStruct((BATCH, OUT), jnp.float32),
    )(partials, ptc, W1, row(b1), row(g1), row(bt1),
      W2, row(b2), row(g2), row(bt2),
      W3.reshape(1, 2 * EMB), b3.reshape(1, 1))


def kernel(x_0, x_0_batch, num_cells_0, W1, b1, g1, bt1, W2, b2, g2, bt2, W3, b3):
    idx = jnp.squeeze(x_0_batch).astype(jnp.int32)
    partials = _make_sc_segment_sum()(x_0, idx)
    ptc = _tc_tail_partial(x_0, idx.reshape(N // TCHUNK, 1, TCHUNK))
    return _head(partials, ptc, W1, b1, g1, bt1, W2, b2, g2, bt2, W3, b3)


# speedup vs baseline: 1.1272x; 1.1272x over previous
import functools

import jax
import jax.numpy as jnp
from jax import lax
from jax.experimental import pallas as pl
from jax.experimental.pallas import tpu as pltpu
from jax.experimental.pallas import tpu_sc as plsc

N = 100000
EMB = 128
BATCH = 1024
OUT = 1

TCHUNK = 800
G_TC = 21
N_SC = N - G_TC * TCHUNK
TC_OFF = N_SC // TCHUNK

NC = 2
NS = 16
NW = NC * NS
CHUNK = 128
NBUF = 6
ROUNDS = 21
ACC_ROWS = 1152
ZROWS = ACC_ROWS // NS

_BN_INV = 0.9999950000374997


@functools.cache
def _make_sc_segment_sum():
    mesh = plsc.VectorSubcoreMesh(
        core_axis_name="c", subcore_axis_name="s",
        num_cores=NC, num_subcores=NS)
    return pl.kernel(
        _sc_segment_sum_body,
        out_type=jax.ShapeDtypeStruct((NC, BATCH, EMB), jnp.float32),
        mesh=mesh,
        scratch_types=[
            pltpu.VMEM((ROUNDS, CHUNK), jnp.int32),
            [pltpu.VMEM((CHUNK, EMB), jnp.float32) for _ in range(NBUF)],
            pltpu.VMEM((ZROWS, EMB), jnp.float32),
            pltpu.VMEM_SHARED((ACC_ROWS, EMB), jnp.float32),
            [pltpu.SemaphoreType.DMA for _ in range(NBUF)],
            [pltpu.SemaphoreType.DMA for _ in range(NBUF)],
            pltpu.SemaphoreType.DMA,
        ],
    )


def _sc_segment_sum_body(x_hbm, idx_hbm, out_hbm,
                         idx_v, bufs, bounce_v, acc_sh, lsems, ssems, isem):
    c = lax.axis_index("c")
    s = lax.axis_index("s")
    w = c * NS + s

    def row_start(r):
        return (w * ROUNDS + r) * CHUNK

    def cond_full(r):
        return row_start(r) + CHUNK <= N_SC

    def load_full(r):
        return pltpu.make_async_copy(
            x_hbm.at[pl.ds(row_start(r), CHUNK)], bufs[r % NBUF],
            lsems[r % NBUF])

    def scat(r):
        return pltpu.make_async_copy(
            bufs[r % NBUF], acc_sh.at[idx_v.at[r]], ssems[r % NBUF])

    def issue_load(r):
        @pl.when(cond_full(r))
        def _():
            load_full(r).start()

    def idx_full(r):
        return pltpu.make_async_copy(
            idx_hbm.at[pl.ds(row_start(r), CHUNK)], idx_v.at[r], isem)

    for k in range(NBUF - 1):
        issue_load(k)

    for r in range(ROUNDS):
        @pl.when(cond_full(r))
        def _():
            idx_full(r).start()

    zvec = jnp.zeros((16,), jnp.float32)

    def _zero_row(i, _):
        for j in range(EMB // 16):
            bounce_v[i, pl.ds(j * 16, 16)] = zvec
        return 0

    lax.fori_loop(0, ZROWS, _zero_row, 0)
    pltpu.sync_copy(bounce_v, acc_sh.at[pl.ds(s * ZROWS, ZROWS)])

    for r in range(ROUNDS):
        @pl.when(cond_full(r))
        def _():
            idx_full(r).wait()

    plsc.subcore_barrier()

    for r in range(ROUNDS):
        @pl.when(cond_full(r))
        def _full():
            load_full(r).wait()
            scat(r).start(add=True)

        nxt = r + NBUF - 1
        if nxt < ROUNDS:
            prev = nxt - NBUF
            if prev >= 0:
                @pl.when(cond_full(prev))
                def _drain():
                    scat(prev).wait()
            issue_load(nxt)

    for r in range(max(0, ROUNDS - NBUF), ROUNDS):
        @pl.when(cond_full(r))
        def _drain_tail():
            scat(r).wait()

    plsc.subcore_barrier()

    wrows = BATCH // NS
    pltpu.sync_copy(acc_sh.at[pl.ds(s * wrows, wrows)],
                    bounce_v.at[pl.ds(0, wrows)])
    pltpu.sync_copy(bounce_v.at[pl.ds(0, wrows)],
                    out_hbm.at[c].at[pl.ds(s * wrows, wrows)])


def _tc_tail_body(x_ref, idx_ref, o_ref):
    g = pl.program_id(0)

    @pl.when(g == 0)
    def _():
        o_ref[...] = jnp.zeros_like(o_ref)

    ids = idx_ref[0, 0, :]
    oht = (lax.broadcasted_iota(jnp.int32, (BATCH, TCHUNK), 0)
           == ids[None, :]).astype(jnp.float32)
    o_ref[...] += jnp.dot(oht, x_ref[...], preferred_element_type=jnp.float32)


def _tc_tail_partial(x_0, idx3):
    return pl.pallas_call(
        _tc_tail_body,
        grid=(G_TC,),
        in_specs=[
            pl.BlockSpec((TCHUNK, EMB), lambda g: (TC_OFF + g, 0)),
            pl.BlockSpec((1, 1, TCHUNK), lambda g: (TC_OFF + g, 0, 0)),
        ],
        out_specs=pl.BlockSpec((BATCH, EMB), lambda g: (0, 0)),
        out_shape=jax.ShapeDtypeStruct((BATCH, EMB), jnp.float32),
    )(x_0, idx3)


def _head_body(p_ref, ptc_ref, w1, b1, g1, bt1, w2, b2, g2, bt2, w3t, b3, o_ref):
    pooled = p_ref[0] + p_ref[1] + ptc_ref[...]
    h = jnp.dot(pooled, w1[...], preferred_element_type=jnp.float32) + b1[...]
    h = jnp.maximum(h * (g1[...] * _BN_INV) + bt1[...], 0.0)
    h = jnp.dot(h, w2[...], preferred_element_type=jnp.float32) + b2[...]
    h = jnp.maximum(h * (g2[...] * _BN_INV) + bt2[...], 0.0)
    o_ref[...] = jnp.sum(h * w3t[...], axis=1, keepdims=True) + b3[...]


def _head(partials, ptc, W1, b1, g1, bt1, W2, b2, g2, bt2, W3, b3):
    row = lambda v: v.reshape(1, -1)
    return pl.pallas_call(
        _head_body,
        out_shape=jax.ShapeDtypeStruct((BATCH, OUT), jnp.float32),
    )(partials, ptc, W1, row(b1), row(g1), row(bt1),
      W2, row(b2), row(g2), row(bt2),
      W3.reshape(1, 2 * EMB), b3.reshape(1, 1))


def kernel(x_0, x_0_batch, num_cells_0, W1, b1, g1, bt1, W2, b2, g2, bt2, W3, b3):
    idx = jnp.squeeze(x_0_batch).astype(jnp.int32)
    partials = _make_sc_segment_sum()(x_0, idx)
    ptc = _tc_tail_partial(x_0, idx.reshape(N // TCHUNK, 1, TCHUNK))
    return _head(partials, ptc, W1, b1, g1, bt1, W2, b2, g2, bt2, W3, b3)
